# baseline (speedup 1.0000x reference)
import functools
import math

import jax
import jax.numpy as jnp
from jax import lax
from jax.experimental import pallas as pl
from jax.experimental.pallas import tpu as pltpu
from jax.experimental.pallas import tpu_sc as plsc

_F32 = jnp.float32


def _rs(n):
    return 1.0 / math.sqrt(float(n))


def _pa_body(D, QK, ni_ref, na_ref, wl1_ref, w3f_ref, whq8_ref, wd84_ref,
             wsc2_ref, feat_ref, u_ref, qd_ref, sc_ref):
    ni = ni_ref[...]
    feat = jnp.dot(ni, wl1_ref[...], preferred_element_type=_F32) * _rs(D)
    feat_ref[...] = feat
    u_ref[...] = jnp.dot(feat, w3f_ref[...], preferred_element_type=_F32)
    q8 = jnp.dot(feat, whq8_ref[...], preferred_element_type=_F32) * _rs(D)
    qd_ref[...] = jnp.dot(q8, wd84_ref[...],
                          preferred_element_type=_F32) * (1.0 / QK)
    sc_ref[...] = jnp.dot(ni * na_ref[...], wsc2_ref[...],
                          preferred_element_type=_F32) * _rs(D)


def _pass_a(ni, na, wl1, w3f, whq8, wd84, wsc2, QK, BN):
    N, D = ni.shape
    grid = (N // BN,)
    full = lambda shp: pl.BlockSpec(shp, lambda i: (0, 0))
    return pl.pallas_call(
        functools.partial(_pa_body, D, QK),
        grid=grid,
        in_specs=[
            pl.BlockSpec((BN, D), lambda i: (i, 0)),
            pl.BlockSpec((BN, 1), lambda i: (i, 0)),
            full((D, D)),
            full((D, 128)),
            full((D, 8)),
            full((8, QK)),
            full((D, D)),
        ],
        out_specs=[
            pl.BlockSpec((BN, D), lambda i: (i, 0)),
            pl.BlockSpec((BN, 128), lambda i: (i, 0)),
            pl.BlockSpec((BN, QK), lambda i: (i, 0)),
            pl.BlockSpec((BN, D), lambda i: (i, 0)),
        ],
        out_shape=[
            jax.ShapeDtypeStruct((N, D), _F32),
            jax.ShapeDtypeStruct((N, 128), _F32),
            jax.ShapeDtypeStruct((N, QK), _F32),
            jax.ShapeDtypeStruct((N, D), _F32),
        ],
    )(ni, na, wl1, w3f, whq8, wd84, wsc2)


def _pc1_body(SCAL, HID, es8_ref, w0_ref, w1_ref, w2_ref, h_ref):
    def layer(x, w_ref, fan_in):
        y = jnp.dot(x, w_ref[...], preferred_element_type=_F32) * _rs(fan_in)
        return y * jax.nn.sigmoid(y)
    h = layer(es8_ref[...], w0_ref, SCAL)
    h = layer(h, w1_ref, HID)
    h = layer(h, w2_ref, HID)
    h_ref[...] = h


def _pass_c1(es8, w0kv, w1kv, w2kv, SCAL, HID, BE8):
    E8 = es8.shape[0]
    grid = (E8 // BE8,)
    blk = pl.BlockSpec((BE8, 128), lambda i: (i, 0))
    full = pl.BlockSpec((128, 128), lambda i: (0, 0))
    return pl.pallas_call(
        functools.partial(_pc1_body, SCAL, HID),
        grid=grid,
        in_specs=[blk, full, full, full],
        out_specs=blk,
        out_shape=jax.ShapeDtypeStruct((E8, 128), _F32),
    )(es8, w0kv, w1kv, w2kv)


def _pc_body(D, HID, QK, h_ref, usrc_ref, ea_ref, v3_ref, r_ref, s4_ref,
             m0_ref, t4_ref):
    h = h_ref[...]
    h3k = h[:, :HID]
    h3v = h[:, HID:2 * HID]
    wv = jnp.dot(h3v, v3_ref[...], preferred_element_type=_F32) * _rs(HID)
    ea = ea_ref[...]
    m0_ref[...] = wv * ea
    s = usrc_ref[...] * jnp.dot(h3k, r_ref[...], preferred_element_type=_F32)
    t4 = jnp.dot(s, s4_ref[...], preferred_element_type=_F32)
    t4_ref[...] = t4 * ea * (_rs(D) * _rs(HID))


def _pass_c(h3kv, usrc, ea, v3, R, S4, D, HID, QK, BE):
    E = h3kv.shape[0]
    HQ = usrc.shape[1]
    grid = (E // BE,)
    full2 = lambda a, b: pl.BlockSpec((a, b), lambda i: (0, 0))
    return pl.pallas_call(
        functools.partial(_pc_body, D, HID, QK),
        grid=grid,
        in_specs=[
            pl.BlockSpec((BE, 2 * HID), lambda i: (i, 0)),
            pl.BlockSpec((BE, HQ), lambda i: (i, 0)),
            pl.BlockSpec((BE, 1), lambda i: (i, 0)),
            full2(HID, D),
            full2(HID, HQ), full2(HQ, QK),
        ],
        out_specs=[
            pl.BlockSpec((BE, D), lambda i: (i, 0)),
            pl.BlockSpec((BE, QK), lambda i: (i, 0)),
        ],
        out_shape=[
            jax.ShapeDtypeStruct((E, D), _F32),
            jax.ShapeDtypeStruct((E, QK), _F32),
        ],
    )(h3kv, usrc, ea, v3, R, S4)


def _pg_body(D, sc_ref, nf0_ref, nf1_ref, z0_ref, z1_ref, csh_ref, wl2_ref,
             out_ref):
    cs = csh_ref[...]
    s = jnp.exp(cs - jnp.max(cs))
    s0 = s[0:1, 0:1]
    s1 = s[1:2, 0:1]
    zs = s0 * z0_ref[...] + s1 * z1_ref[...]
    zs = jnp.where(zs == 0.0, jnp.ones_like(zs), zs)
    nf = (s0 * nf0_ref[...] + s1 * nf1_ref[...]) / zs
    out_ref[...] = sc_ref[...] + jnp.dot(
        nf, wl2_ref[...], preferred_element_type=_F32) * _rs(D)


def _pass_g(sc, nf0, nf1, z0, z1, csh, wl2, BN):
    N, D = sc.shape
    grid = (N // BN,)
    blk = pl.BlockSpec((BN, D), lambda i: (i, 0))
    col = pl.BlockSpec((BN, 1), lambda i: (i, 0))
    return pl.pallas_call(
        functools.partial(_pg_body, D),
        grid=grid,
        in_specs=[blk, blk, blk, col, col,
                  pl.BlockSpec((2, 16), lambda i: (0, 0)),
                  pl.BlockSpec((D, D), lambda i: (0, 0))],
        out_specs=blk,
        out_shape=jax.ShapeDtypeStruct((N, D), _F32),
    )(sc, nf0, nf1, z0, z1, csh, wl2)


_CH = 128
_NW = 32
_SC_PARAMS = pltpu.CompilerParams(needs_layout_passes=False)


def _worker_id():
    return lax.axis_index("s") * 2 + lax.axis_index("c")


def _n_chunks(total_chunks, wid):
    base = total_chunks // _NW
    rem = total_chunks % _NW
    return jnp.where(wid < rem, base + 1, base).astype(jnp.int32)


def _pass_b(u, src):
    N, HQ = u.shape
    E = src.shape[0]
    total_chunks = E // _CH
    mesh = plsc.VectorSubcoreMesh(core_axis_name="c", subcore_axis_name="s")

    @functools.partial(
        pl.kernel,
        out_type=jax.ShapeDtypeStruct((E * 32,), _F32),
        mesh=mesh,
        compiler_params=_SC_PARAMS,
        scratch_types=[
            pltpu.VMEM((_CH,), jnp.int32),
            pltpu.VMEM((_CH, HQ), _F32),
            pltpu.VMEM((_CH * 32,), _F32),
            pltpu.SemaphoreType.DMA,
            pltpu.SemaphoreType.DMA,
        ],
    )
    def kfn(u_hbm, src_hbm, usrc_hbm, sidx, urows, ucomp, sem1, sem2):
        wid = _worker_id()
        nchunks = _n_chunks(total_chunks, wid)

        def body(i, carry):
            base = (wid + _NW * i) * _CH
            pltpu.sync_copy(src_hbm.at[pl.ds(base, _CH)], sidx)
            pltpu.async_copy(u_hbm.at[sidx], urows, sem1).wait()

            def comp(jj, carry2):
                for uu in range(4):
                    j = jj * 4 + uu
                    ucomp[pl.ds(j * 32, 16)] = urows[j, pl.ds(0, 16)]
                    ucomp[pl.ds(j * 32 + 16, 16)] = urows[j, pl.ds(16, 16)]
                return carry2
            lax.fori_loop(0, _CH // 4, comp, 0)

            pltpu.async_copy(ucomp, usrc_hbm.at[pl.ds(base * 32, _CH * 32)],
                             sem2).wait()
            return carry

        lax.fori_loop(0, nchunks, body, 0)

    return kfn(u, src)


def _pass_f(feat, src, dst, t4f, m0, qds, NP):
    N, D = feat.shape
    E = src.shape[0]
    total_chunks = E // _CH
    rows_per_tile = NP // 16
    nvec = D // 16
    mesh = plsc.VectorSubcoreMesh(core_axis_name="c", subcore_axis_name="s")

    @functools.partial(
        pl.kernel,
        out_type=(
            jax.ShapeDtypeStruct((2, NP, D), _F32),
            jax.ShapeDtypeStruct((2, NP), _F32),
            jax.ShapeDtypeStruct((32,), _F32),
            jax.ShapeDtypeStruct((E,), _F32),
        ),
        mesh=mesh,
        compiler_params=_SC_PARAMS,
        scratch_types=[
            pltpu.VMEM((N,), _F32),
            pltpu.VMEM((_CH,), jnp.int32),
            pltpu.VMEM((_CH,), jnp.int32),
            pltpu.VMEM((_CH * 4,), _F32),
            pltpu.VMEM((_CH,), _F32),
            pltpu.VMEM((_CH,), _F32),
            pltpu.VMEM((16,), _F32),
            pltpu.VMEM((_CH, D), _F32),
            pltpu.VMEM((_CH, D), _F32),
            pltpu.VMEM_SHARED((NP, D), _F32),
            pltpu.VMEM_SHARED((NP,), _F32),
            pltpu.VMEM_SHARED((16, 16), _F32),
            pltpu.SemaphoreType.DMA,
            pltpu.SemaphoreType.DMA,
            pltpu.SemaphoreType.DMA,
            pltpu.SemaphoreType.DMA,
            pltpu.SemaphoreType.DMA,
        ],
    )
    def kfn(feat_hbm, src_hbm, dst_hbm, t4f_hbm, m0_hbm,
            q0_hbm, q1_hbm, q2_hbm, q3_hbm,
            nf_hbm, z_hbm, csh_hbm, x_hbm,
            qdt, sidx, didx, tc, xc, ec, mxv, featg, m0c,
            nfsp, zsp, msp, semA, semB, semC, semD, semE):
        cid = lax.axis_index("c")
        sid = lax.axis_index("s")
        wid = _worker_id()
        nchunks = _n_chunks(total_chunks, wid)

        mx = jnp.full((16,), -jnp.inf, _F32)
        for k, q_hbm in enumerate([q0_hbm, q1_hbm, q2_hbm, q3_hbm]):
            pltpu.sync_copy(q_hbm, qdt)

            def xbody(i, mxc):
                base = (wid + _NW * i) * _CH
                pltpu.sync_copy(dst_hbm.at[pl.ds(base, _CH)], didx)
                pltpu.sync_copy(t4f_hbm.at[pl.ds(base * 4, _CH * 4)], tc)
                if k > 0:
                    pltpu.sync_copy(x_hbm.at[pl.ds(base, _CH)], xc)
                for v in range(_CH // 16):
                    sl = pl.ds(v * 16, 16)
                    lane = lax.iota(jnp.int32, 16) * 4 + (v * 64 + k)
                    tv = plsc.load_gather(tc, [lane])
                    qv = plsc.load_gather(qdt, [didx[sl]])
                    if k == 0:
                        acc = tv * qv
                    else:
                        acc = xc[sl] + tv * qv
                    xc[sl] = acc
                    if k == 3:
                        mxc = jnp.maximum(mxc, acc)
                pltpu.sync_copy(xc, x_hbm.at[pl.ds(base, _CH)])
                return mxc

            mx = lax.fori_loop(0, nchunks, xbody, mx)

        mxv[...] = mx
        pltpu.sync_copy(mxv, msp.at[sid])
        plsc.subcore_barrier()
        pltpu.sync_copy(msp.at[0], mxv)
        gm = mxv[...]
        for t in range(1, 16):
            pltpu.sync_copy(msp.at[t], mxv)
            gm = jnp.maximum(gm, mxv[...])
        cshift = jnp.max(gm, axis=0)
        gmv = jnp.full((16,), cshift, _F32)

        @pl.when(sid == 0)
        def _():
            mxv[...] = gmv
            pltpu.sync_copy(mxv, csh_hbm.at[pl.ds(cid * 16, 16)])

        def zfill(i, carry):
            for v in range(nvec):
                featg[i, pl.ds(v * 16, 16)] = jnp.zeros((16,), _F32)
            return carry
        lax.fori_loop(0, _CH, zfill, 0)

        def zrows(i, carry):
            pltpu.sync_copy(
                featg, nfsp.at[pl.ds(sid * rows_per_tile + i * _CH, _CH)])
            return carry
        lax.fori_loop(0, rows_per_tile // _CH, zrows, 0)
        for v in range(rows_per_tile // 16):
            qdt[pl.ds(v * 16, 16)] = jnp.zeros((16,), _F32)
        pltpu.sync_copy(qdt.at[pl.ds(0, rows_per_tile)],
                        zsp.at[pl.ds(sid * rows_per_tile, rows_per_tile)])
        plsc.subcore_barrier()

        def body(i, carry):
            base = (wid + _NW * i) * _CH
            cp_s = pltpu.async_copy(src_hbm.at[pl.ds(base, _CH)], sidx, semA)
            cp_d = pltpu.async_copy(dst_hbm.at[pl.ds(base, _CH)], didx, semB)
            cp_x = pltpu.async_copy(x_hbm.at[pl.ds(base, _CH)], xc, semC)
            cp_m = pltpu.async_copy(m0_hbm.at[pl.ds(base, _CH)], m0c, semD)
            cp_s.wait()
            cp_g = pltpu.async_copy(feat_hbm.at[sidx], featg, semE)
            cp_x.wait()
            for v in range(_CH // 16):
                sl = pl.ds(v * 16, 16)
                ec[sl] = jnp.exp(xc[sl] - gmv)
            cp_d.wait()
            pltpu.sync_copy(ec, zsp.at[didx], add=True)
            cp_m.wait()
            cp_g.wait()

            def rows(jj, carry2):
                for uu in range(4):
                    j = jj * 4 + uu
                    av = plsc.load_gather(
                        ec, [jnp.full((16,), j, jnp.int32)])
                    for v in range(nvec):
                        sl = pl.ds(v * 16, 16)
                        m0c[j, sl] = m0c[j, sl] * featg[j, sl] * av
                return carry2
            lax.fori_loop(0, _CH // 4, rows, 0)

            pltpu.sync_copy(m0c, nfsp.at[didx], add=True)
            return carry

        lax.fori_loop(0, nchunks, body, 0)
        plsc.subcore_barrier()

        def dbody(i, carry):
            r0 = sid * rows_per_tile + i * _CH
            pltpu.sync_copy(nfsp.at[pl.ds(r0, _CH)], featg)
            pltpu.sync_copy(featg, nf_hbm.at[cid, pl.ds(r0, _CH)])
            return carry
        lax.fori_loop(0, rows_per_tile // _CH, dbody, 0)
        pltpu.sync_copy(zsp.at[pl.ds(sid * rows_per_tile, rows_per_tile)],
                        qdt.at[pl.ds(0, rows_per_tile)])
        pltpu.sync_copy(qdt.at[pl.ds(0, rows_per_tile)],
                        z_hbm.at[cid, pl.ds(sid * rows_per_tile,
                                            rows_per_tile)])

    return kfn(feat, src, dst, t4f, m0, *qds)


def kernel(node_input, node_attr, edge_src, edge_dst, edge_attr, edge_scalars,
           W_sc, W_lin1, W_hq, W_dot, W_lin2,
           fck_W0, fck_W1, fck_W2, fck_W3,
           fc_W0, fc_W1, fc_W2, fc_W3):
    N, D = node_input.shape
    E = edge_src.shape[0]
    QK = W_hq.shape[1]
    HID = fck_W0.shape[1]
    SCAL = edge_scalars.shape[1]
    NP = ((N + 1023) // 1024) * 1024
    BN = 2000 if N % 2000 == 0 else 8
    BE = 4000 if E % 4000 == 0 else 128

    w3f = jnp.transpose(fck_W3.reshape(HID, D, QK), (1, 0, 2)).reshape(
        D, HID * QK)
    w3f = jnp.pad(w3f, ((0, 0), (0, 128 - HID * QK)))
    whq8 = jnp.pad(W_hq, ((0, 0), (0, 8 - QK)))
    wd84 = jnp.pad(W_dot[:, :, 0], ((0, 8 - QK), (0, 0)))
    wsc2 = W_sc[:, 0, :]
    R = jnp.repeat(jnp.eye(HID, dtype=_F32), QK, axis=1)
    S4 = jnp.tile(jnp.eye(QK, dtype=_F32), (HID, 1))
    src = edge_src.astype(jnp.int32)
    dst = edge_dst.astype(jnp.int32)
    b0 = jnp.concatenate([fck_W0, fc_W0], axis=1)
    b1 = jnp.concatenate([
        jnp.concatenate([fck_W1, jnp.zeros((HID, HID), _F32)], axis=1),
        jnp.concatenate([jnp.zeros((HID, HID), _F32), fc_W1], axis=1)],
        axis=0)
    b2 = jnp.concatenate([
        jnp.concatenate([fck_W2, jnp.zeros((HID, HID), _F32)], axis=1),
        jnp.concatenate([jnp.zeros((HID, HID), _F32), fc_W2], axis=1)],
        axis=0)
    eye8 = jnp.eye(128 // SCAL, dtype=_F32)
    w0kv = jnp.kron(eye8, b0)
    w1kv = jnp.kron(eye8, b1)
    w2kv = jnp.kron(eye8, b2)
    es8 = edge_scalars.reshape(E * SCAL // 128, 128)

    feat, u, qd4, sc = _pass_a(node_input, node_attr, W_lin1, w3f, whq8,
                               wd84, wsc2, QK, BN)
    usrc = _pass_b(u, src).reshape(E, 32)
    h3kv8 = _pass_c1(es8, w0kv, w1kv, w2kv, SCAL, HID, 2000)
    h3kv = h3kv8.reshape(E, 2 * HID)
    m0, t4 = _pass_c(h3kv, usrc, edge_attr, fc_W3, R, S4, D, HID, QK, BE)
    qds = [qd4[:, k].reshape(N) for k in range(QK)]
    t4f = t4.reshape(E * QK)
    nf2, z2, csh, _x = _pass_f(feat, src, dst, t4f, m0, qds, NP)
    z0 = z2[0, :N].reshape(N, 1)
    z1 = z2[1, :N].reshape(N, 1)
    out = _pass_g(sc, nf2[0, :N, :], nf2[1, :N, :], z0, z1,
                  csh.reshape(2, 16), W_lin2, BN)
    return out


# speedup vs baseline: 1.2339x; 1.2339x over previous
import functools
import math

import jax
import jax.numpy as jnp
from jax import lax
from jax.experimental import pallas as pl
from jax.experimental.pallas import tpu as pltpu
from jax.experimental.pallas import tpu_sc as plsc

_F32 = jnp.float32


def _rs(n):
    return 1.0 / math.sqrt(float(n))


def _silu_layer(x, w_ref):
    w = w_ref[...]
    y = jnp.dot(x, w, preferred_element_type=_F32) * _rs(w.shape[0])
    return y * jax.nn.sigmoid(y)


def _pa_body(D, QK, ni_ref, na_ref, wl1_ref, w3f_ref, whq8_ref, wd816_ref,
             wsc2_ref, feat_ref, u_ref, qd_ref, sc_ref):
    ni = ni_ref[...]
    feat = jnp.dot(ni, wl1_ref[...], preferred_element_type=_F32) * _rs(D)
    feat_ref[...] = feat
    u_ref[...] = jnp.dot(feat, w3f_ref[...], preferred_element_type=_F32)
    q8 = jnp.dot(feat, whq8_ref[...], preferred_element_type=_F32) * _rs(D)
    qd_ref[...] = jnp.dot(q8, wd816_ref[...],
                          preferred_element_type=_F32) * (1.0 / QK)
    sc_ref[...] = jnp.dot(ni * na_ref[...], wsc2_ref[...],
                          preferred_element_type=_F32) * _rs(D)


def _pass_a(ni, na, wl1, w3f, whq8, wd816, wsc2, BN):
    N, D = ni.shape
    QK = 4
    grid = (N // BN,)
    full = lambda shp: pl.BlockSpec(shp, lambda i: (0, 0))
    return pl.pallas_call(
        functools.partial(_pa_body, D, QK),
        grid=grid,
        in_specs=[
            pl.BlockSpec((BN, D), lambda i: (i, 0)),
            pl.BlockSpec((BN, 1), lambda i: (i, 0)),
            full((D, D)),
            full((D, 128)),
            full((D, 8)),
            full((8, 128)),
            full((D, D)),
        ],
        out_specs=[
            pl.BlockSpec((BN, D), lambda i: (i, 0)),
            pl.BlockSpec((BN, 128), lambda i: (i, 0)),
            pl.BlockSpec((BN, 128), lambda i: (i, 0)),
            pl.BlockSpec((BN, D), lambda i: (i, 0)),
        ],
        out_shape=[
            jax.ShapeDtypeStruct((N, D), _F32),
            jax.ShapeDtypeStruct((N, 128), _F32),
            jax.ShapeDtypeStruct((N, 128), _F32),
            jax.ShapeDtypeStruct((N, D), _F32),
        ],
    )(ni, na, wl1, w3f, whq8, wd816, wsc2)


def _pc1_body(SCAL, HID, es8_ref, w0_ref, w1_ref, w2_ref, h_ref):
    def layer(x, w_ref, fan_in):
        y = jnp.dot(x, w_ref[...], preferred_element_type=_F32) * _rs(fan_in)
        return y * jax.nn.sigmoid(y)
    h = layer(es8_ref[...], w0_ref, SCAL)
    h = layer(h, w1_ref, HID)
    h = layer(h, w2_ref, HID)
    h_ref[...] = h


def _pass_c1(es8, w0kv, w1kv, w2kv, SCAL, HID, BE8):
    E8 = es8.shape[0]
    grid = (E8 // BE8,)
    blk = pl.BlockSpec((BE8, 128), lambda i: (i, 0))
    full = pl.BlockSpec((128, 128), lambda i: (0, 0))
    return pl.pallas_call(
        functools.partial(_pc1_body, SCAL, HID),
        grid=grid,
        in_specs=[blk, full, full, full],
        out_specs=blk,
        out_shape=jax.ShapeDtypeStruct((E8, 128), _F32),
    )(es8, w0kv, w1kv, w2kv)


def _pc_body(D, HID, h_ref, usrc_ref, qdd_ref, ea_ref,
             v3_ref, r_ref, s_ref, m0_ref, x_ref, xmax_ref, macc_ref):
    h = h_ref[...]
    h3k = h[:, :HID]
    h3v = h[:, HID:2 * HID]
    wv = jnp.dot(h3v, v3_ref[...], preferred_element_type=_F32) * _rs(HID)
    ea = ea_ref[...]
    m0_ref[...] = wv * ea
    s = usrc_ref[...] * jnp.dot(h3k, r_ref[...], preferred_element_type=_F32)
    t16 = jnp.dot(s, s_ref[...], preferred_element_type=_F32)
    x = jnp.sum(t16 * qdd_ref[...], axis=1, keepdims=True)
    x = x * ea * (_rs(D) * _rs(HID))
    x_ref[...] = x

    @pl.when(pl.program_id(0) == 0)
    def _():
        macc_ref[0] = jnp.float32(-jnp.inf)

    m = jnp.maximum(macc_ref[0], jnp.max(x))
    macc_ref[0] = m
    xmax_ref[...] = jnp.full((1, 128), m, dtype=_F32)


def _pass_c(h3kv, usrc, qdd16, ea, v3, R, S16, D, HID, BE):
    E = h3kv.shape[0]
    grid = (E // BE,)
    full2 = lambda a, b: pl.BlockSpec((a, b), lambda i: (0, 0))
    return pl.pallas_call(
        functools.partial(_pc_body, D, HID),
        grid=grid,
        in_specs=[
            pl.BlockSpec((BE, 16), lambda i: (i, 0)),
            pl.BlockSpec((BE, 128), lambda i: (i, 0)),
            pl.BlockSpec((BE, 128), lambda i: (i, 0)),
            pl.BlockSpec((BE, 1), lambda i: (i, 0)),
            full2(HID, D),
            full2(HID, 128), full2(128, 128),
        ],
        out_specs=[
            pl.BlockSpec((BE, D), lambda i: (i, 0)),
            pl.BlockSpec((BE, 1), lambda i: (i, 0)),
            pl.BlockSpec((1, 128), lambda i: (0, 0)),
        ],
        out_shape=[
            jax.ShapeDtypeStruct((E, D), _F32),
            jax.ShapeDtypeStruct((E, 1), _F32),
            jax.ShapeDtypeStruct((1, 128), _F32),
        ],
        scratch_shapes=[pltpu.SMEM((1,), _F32)],
    )(h3kv, usrc, qdd16, ea, v3, R, S16)


def _pg_body(D, sc_ref, nf0_ref, nf1_ref, z0_ref, z1_ref, wl2_ref, out_ref):
    zs = z0_ref[...] + z1_ref[...]
    zs = jnp.where(zs == 0.0, jnp.ones_like(zs), zs)
    nf = (nf0_ref[...] + nf1_ref[...]) / zs
    out_ref[...] = sc_ref[...] + jnp.dot(
        nf, wl2_ref[...], preferred_element_type=_F32) * _rs(D)


def _pass_g(sc, nf0, nf1, z0, z1, wl2, BN):
    N, D = sc.shape
    grid = (N // BN,)
    blk = pl.BlockSpec((BN, D), lambda i: (i, 0))
    col = pl.BlockSpec((BN, 1), lambda i: (i, 0))
    return pl.pallas_call(
        functools.partial(_pg_body, D),
        grid=grid,
        in_specs=[blk, blk, blk, col, col,
                  pl.BlockSpec((D, D), lambda i: (0, 0))],
        out_specs=blk,
        out_shape=jax.ShapeDtypeStruct((N, D), _F32),
    )(sc, nf0, nf1, z0, z1, wl2)


_CH = 128
_NW = 32
_SC_PARAMS = pltpu.CompilerParams(needs_layout_passes=False)


def _worker_id():
    return lax.axis_index("s") * 2 + lax.axis_index("c")


def _n_chunks(total_chunks, wid):
    base = total_chunks // _NW
    rem = total_chunks % _NW
    return jnp.where(wid < rem, base + 1, base).astype(jnp.int32)


def _pass_b(u, qd, src, dst):
    N, HQ = u.shape
    E = src.shape[0]
    total_chunks = E // _CH
    mesh = plsc.VectorSubcoreMesh(core_axis_name="c", subcore_axis_name="s")

    @functools.partial(
        pl.kernel,
        out_type=(
            jax.ShapeDtypeStruct((E, HQ), _F32),
            jax.ShapeDtypeStruct((E, HQ), _F32),
        ),
        mesh=mesh,
        compiler_params=_SC_PARAMS,
        scratch_types=[
            pltpu.VMEM((2, _CH), jnp.int32),
            pltpu.VMEM((2, _CH), jnp.int32),
            pltpu.VMEM((2, _CH, HQ), _F32),
            pltpu.VMEM((2, _CH, HQ), _F32),
        ] + [pltpu.SemaphoreType.DMA] * 8,
    )
    def kfn(u_hbm, qd_hbm, src_hbm, dst_hbm, usrc_hbm, qdd_hbm,
            sidx, didx, urows, qrows,
            ss0, ss1, sd0, sd1, su0, su1, sq0, sq1):
        wid = _worker_id()
        nchunks = _n_chunks(total_chunks, wid)
        ssem = [ss0, ss1]
        dsem = [sd0, sd1]
        usem = [su0, su1]
        qsem = [sq0, sq1]

        def pair(jj, carry):
            idx_cp = [None, None]
            for b in range(2):
                ci = jj * 2 + b

                @pl.when(ci < nchunks)
                def _():
                    base = (wid + _NW * ci) * _CH
                    pltpu.async_copy(src_hbm.at[pl.ds(base, _CH)],
                                     sidx.at[b], ssem[b])
                    pltpu.async_copy(dst_hbm.at[pl.ds(base, _CH)],
                                     didx.at[b], dsem[b])
            gat_cp = [None, None]
            for b in range(2):
                ci = jj * 2 + b

                @pl.when(ci < nchunks)
                def _():
                    pltpu.make_async_copy(src_hbm.at[pl.ds(0, _CH)],
                                          sidx.at[b], ssem[b]).wait()
                    pltpu.make_async_copy(dst_hbm.at[pl.ds(0, _CH)],
                                          didx.at[b], dsem[b]).wait()
                    pltpu.async_copy(u_hbm.at[sidx.at[b]], urows.at[b],
                                     usem[b])
                    pltpu.async_copy(qd_hbm.at[didx.at[b]], qrows.at[b],
                                     qsem[b])
            for b in range(2):
                ci = jj * 2 + b

                @pl.when(ci < nchunks)
                def _():
                    base = (wid + _NW * ci) * _CH
                    pltpu.make_async_copy(u_hbm.at[pl.ds(0, _CH)],
                                          urows.at[b], usem[b]).wait()
                    pltpu.make_async_copy(qd_hbm.at[pl.ds(0, _CH)],
                                          qrows.at[b], qsem[b]).wait()
                    pltpu.async_copy(urows.at[b],
                                     usrc_hbm.at[pl.ds(base, _CH)], usem[b])
                    pltpu.async_copy(qrows.at[b],
                                     qdd_hbm.at[pl.ds(base, _CH)], qsem[b])
            for b in range(2):
                ci = jj * 2 + b

                @pl.when(ci < nchunks)
                def _():
                    base = (wid + _NW * ci) * _CH
                    pltpu.make_async_copy(urows.at[b],
                                          usrc_hbm.at[pl.ds(base, _CH)],
                                          usem[b]).wait()
                    pltpu.make_async_copy(qrows.at[b],
                                          qdd_hbm.at[pl.ds(base, _CH)],
                                          qsem[b]).wait()
            return carry

        lax.fori_loop(0, (total_chunks // _NW + 2) // 2, pair, 0)

    return kfn(u, qd, src, dst)


def _pass_d(x, xmax, dst, NP):
    E = x.shape[0]
    total_chunks = E // _CH
    rows_per_tile = NP // 16
    mesh = plsc.VectorSubcoreMesh(core_axis_name="c", subcore_axis_name="s")

    @functools.partial(
        pl.kernel,
        out_type=(
            jax.ShapeDtypeStruct((E,), _F32),
            jax.ShapeDtypeStruct((2, NP), _F32),
        ),
        mesh=mesh,
        compiler_params=_SC_PARAMS,
        scratch_types=[
            pltpu.VMEM((_CH,), _F32),
            pltpu.VMEM((_CH,), _F32),
            pltpu.VMEM((_CH,), jnp.int32),
            pltpu.VMEM((128,), _F32),
            pltpu.VMEM((rows_per_tile,), _F32),
            pltpu.VMEM_SHARED((NP,), _F32),
        ],
    )
    def kfn(x_hbm, xmax_hbm, dst_hbm, expv_hbm, z_hbm,
            xc, ec, didx, xmv, ztile, zsp):
        cid = lax.axis_index("c")
        sid = lax.axis_index("s")
        wid = _worker_id()
        nchunks = _n_chunks(total_chunks, wid)

        def zbody(i, carry):
            ztile[pl.ds(i * 16, 16)] = jnp.zeros((16,), _F32)
            return carry
        lax.fori_loop(0, rows_per_tile // 16, zbody, 0)
        pltpu.sync_copy(ztile, zsp.at[pl.ds(sid * rows_per_tile,
                                            rows_per_tile)])
        plsc.subcore_barrier()

        pltpu.sync_copy(xmax_hbm, xmv)
        xmax_v = xmv[pl.ds(0, 16)]

        def body(i, carry):
            base = (wid + _NW * i) * _CH
            pltpu.sync_copy(x_hbm.at[pl.ds(base, _CH)], xc)
            pltpu.sync_copy(dst_hbm.at[pl.ds(base, _CH)], didx)
            for v in range(_CH // 16):
                sl = pl.ds(v * 16, 16)
                ec[sl] = jnp.exp(xc[sl] - xmax_v)
            pltpu.sync_copy(ec, expv_hbm.at[pl.ds(base, _CH)])
            pltpu.sync_copy(ec, zsp.at[didx], add=True)
            return carry

        lax.fori_loop(0, nchunks, body, 0)
        plsc.subcore_barrier()

        pltpu.sync_copy(zsp.at[pl.ds(sid * rows_per_tile, rows_per_tile)],
                        ztile)
        pltpu.sync_copy(ztile, z_hbm.at[cid, pl.ds(sid * rows_per_tile,
                                                   rows_per_tile)])

    return kfn(x, xmax, dst)


def _pass_f(feat, src, dst, expv, m0, NP):
    N, D = feat.shape
    E = src.shape[0]
    total_chunks = E // _CH
    rows_per_tile = NP // 16
    nvec = D // 16
    mesh = plsc.VectorSubcoreMesh(core_axis_name="c", subcore_axis_name="s")

    @functools.partial(
        pl.kernel,
        out_type=jax.ShapeDtypeStruct((2, NP, D), _F32),
        mesh=mesh,
        compiler_params=_SC_PARAMS,
        scratch_types=[
            pltpu.VMEM((_CH,), jnp.int32),
            pltpu.VMEM((_CH,), jnp.int32),
            pltpu.VMEM((_CH,), _F32),
            pltpu.VMEM((_CH, D), _F32),
            pltpu.VMEM((_CH, D), _F32),
            pltpu.VMEM_SHARED((NP, D), _F32),
            pltpu.SemaphoreType.DMA,
            pltpu.SemaphoreType.DMA,
            pltpu.SemaphoreType.DMA,
            pltpu.SemaphoreType.DMA,
            pltpu.SemaphoreType.DMA,
        ],
    )
    def kfn(feat_hbm, src_hbm, dst_hbm, expv_hbm, m0_hbm, nf_hbm,
            sidx, didx, ec, featg, m0c, nfsp, semA, semB, semC, semD, semE):
        cid = lax.axis_index("c")
        sid = lax.axis_index("s")
        wid = _worker_id()
        nchunks = _n_chunks(total_chunks, wid)

        def zfill(i, carry):
            for v in range(nvec):
                featg[i, pl.ds(v * 16, 16)] = jnp.zeros((16,), _F32)
            return carry
        lax.fori_loop(0, _CH, zfill, 0)

        def zrows(i, carry):
            pltpu.sync_copy(
                featg, nfsp.at[pl.ds(sid * rows_per_tile + i * _CH, _CH)])
            return carry
        lax.fori_loop(0, rows_per_tile // _CH, zrows, 0)
        plsc.subcore_barrier()

        def body(i, carry):
            base = (wid + _NW * i) * _CH
            cp_s = pltpu.async_copy(src_hbm.at[pl.ds(base, _CH)], sidx, semA)
            cp_d = pltpu.async_copy(dst_hbm.at[pl.ds(base, _CH)], didx, semB)
            cp_e = pltpu.async_copy(expv_hbm.at[pl.ds(base, _CH)], ec, semC)
            cp_m = pltpu.async_copy(m0_hbm.at[pl.ds(base, _CH)], m0c, semD)
            cp_s.wait()
            cp_g = pltpu.async_copy(feat_hbm.at[sidx], featg, semE)
            cp_e.wait()
            cp_m.wait()
            cp_g.wait()

            def rows(jj, carry2):
                for u in range(4):
                    j = jj * 4 + u
                    av = plsc.load_gather(
                        ec, [jnp.full((16,), j, jnp.int32)])
                    for v in range(nvec):
                        sl = pl.ds(v * 16, 16)
                        m0c[j, sl] = m0c[j, sl] * featg[j, sl] * av
                return carry2
            lax.fori_loop(0, _CH // 4, rows, 0)

            cp_d.wait()
            pltpu.sync_copy(m0c, nfsp.at[didx], add=True)
            return carry

        lax.fori_loop(0, nchunks, body, 0)
        plsc.subcore_barrier()

        def dbody(i, carry):
            r0 = sid * rows_per_tile + i * _CH
            pltpu.sync_copy(nfsp.at[pl.ds(r0, _CH)], featg)
            pltpu.sync_copy(featg, nf_hbm.at[cid, pl.ds(r0, _CH)])
            return carry
        lax.fori_loop(0, rows_per_tile // _CH, dbody, 0)

    return kfn(feat, src, dst, expv, m0)


def kernel(node_input, node_attr, edge_src, edge_dst, edge_attr, edge_scalars,
           W_sc, W_lin1, W_hq, W_dot, W_lin2,
           fck_W0, fck_W1, fck_W2, fck_W3,
           fc_W0, fc_W1, fc_W2, fc_W3):
    N, D = node_input.shape
    E = edge_src.shape[0]
    QK = W_hq.shape[1]
    HID = fck_W0.shape[1]
    NP = ((N + 1023) // 1024) * 1024
    BN = 2000 if N % 2000 == 0 else 8
    BE = 4000 if E % 4000 == 0 else 128

    w3f = jnp.transpose(fck_W3.reshape(HID, D, QK), (1, 0, 2)).reshape(
        D, HID * QK)
    w3f = jnp.pad(w3f, ((0, 0), (0, 128 - HID * QK)))
    whq8 = jnp.pad(W_hq, ((0, 0), (0, 8 - QK)))
    wd816 = jnp.pad(W_dot[:, :, 0], ((0, 8 - QK), (0, 128 - QK)))
    wsc2 = W_sc[:, 0, :]
    R = jnp.pad(jnp.repeat(jnp.eye(HID, dtype=_F32), QK, axis=1),
                ((0, 0), (0, 128 - HID * QK)))
    S16 = jnp.pad(jnp.tile(jnp.eye(QK, dtype=_F32), (HID, 1)),
                  ((0, HID * QK - 32 + 96), (0, 128 - QK)))
    src = edge_src.astype(jnp.int32)
    dst = edge_dst.astype(jnp.int32)
    SCAL = edge_scalars.shape[1]
    b0 = jnp.concatenate([fck_W0, fc_W0], axis=1)
    b1 = jnp.concatenate([
        jnp.concatenate([fck_W1, jnp.zeros((HID, HID), _F32)], axis=1),
        jnp.concatenate([jnp.zeros((HID, HID), _F32), fc_W1], axis=1)],
        axis=0)
    b2 = jnp.concatenate([
        jnp.concatenate([fck_W2, jnp.zeros((HID, HID), _F32)], axis=1),
        jnp.concatenate([jnp.zeros((HID, HID), _F32), fc_W2], axis=1)],
        axis=0)
    eye8 = jnp.eye(128 // SCAL, dtype=_F32)
    w0kv = jnp.kron(eye8, b0)
    w1kv = jnp.kron(eye8, b1)
    w2kv = jnp.kron(eye8, b2)
    es8 = edge_scalars.reshape(E * SCAL // 128, 128)

    feat, u, qd16, sc = _pass_a(node_input, node_attr, W_lin1, w3f, whq8,
                                wd816, wsc2, BN)
    usrc, qdd16 = _pass_b(u, qd16, src, dst)
    h3kv8 = _pass_c1(es8, w0kv, w1kv, w2kv, SCAL, HID, 2000)
    h3kv = h3kv8.reshape(E, 2 * HID)
    m0, x, xmax = _pass_c(h3kv, usrc, qdd16, edge_attr, fc_W3, R, S16,
                          D, HID, BE)
    expv, z2 = _pass_d(x.reshape(E), xmax.reshape(128), dst, NP)
    nf2 = _pass_f(feat, src, dst, expv, m0, NP)
    z0 = z2[0, :N].reshape(N, 1)
    z1 = z2[1, :N].reshape(N, 1)
    out = _pass_g(sc, nf2[0, :N, :], nf2[1, :N, :], z0, z1, W_lin2, BN)
    return out


# speedup vs baseline: 1.2974x; 1.0515x over previous
import functools
import math

import jax
import jax.numpy as jnp
from jax import lax
from jax.experimental import pallas as pl
from jax.experimental.pallas import tpu as pltpu
from jax.experimental.pallas import tpu_sc as plsc

_F32 = jnp.float32


def _rs(n):
    return 1.0 / math.sqrt(float(n))


def _silu_layer(x, w_ref):
    w = w_ref[...]
    y = jnp.dot(x, w, preferred_element_type=_F32) * _rs(w.shape[0])
    return y * jax.nn.sigmoid(y)


def _pa_body(D, QK, ni_ref, na_ref, wl1_ref, w3f_ref, whq8_ref, wd816_ref,
             wsc2_ref, feat_ref, u_ref, qd_ref, sc_ref):
    ni = ni_ref[...]
    feat = jnp.dot(ni, wl1_ref[...], preferred_element_type=_F32) * _rs(D)
    feat_ref[...] = feat
    u_ref[...] = jnp.dot(feat, w3f_ref[...], preferred_element_type=_F32)
    q8 = jnp.dot(feat, whq8_ref[...], preferred_element_type=_F32) * _rs(D)
    qd_ref[...] = jnp.dot(q8, wd816_ref[...],
                          preferred_element_type=_F32) * (1.0 / QK)
    sc_ref[...] = jnp.dot(ni * na_ref[...], wsc2_ref[...],
                          preferred_element_type=_F32) * _rs(D)


def _pass_a(ni, na, wl1, w3f, whq8, wd816, wsc2, BN):
    N, D = ni.shape
    QK = 4
    grid = (N // BN,)
    full = lambda shp: pl.BlockSpec(shp, lambda i: (0, 0))
    return pl.pallas_call(
        functools.partial(_pa_body, D, QK),
        grid=grid,
        in_specs=[
            pl.BlockSpec((BN, D), lambda i: (i, 0)),
            pl.BlockSpec((BN, 1), lambda i: (i, 0)),
            full((D, D)),
            full((D, 128)),
            full((D, 8)),
            full((8, 128)),
            full((D, D)),
        ],
        out_specs=[
            pl.BlockSpec((BN, D), lambda i: (i, 0)),
            pl.BlockSpec((BN, 128), lambda i: (i, 0)),
            pl.BlockSpec((BN, 128), lambda i: (i, 0)),
            pl.BlockSpec((BN, D), lambda i: (i, 0)),
        ],
        out_shape=[
            jax.ShapeDtypeStruct((N, D), _F32),
            jax.ShapeDtypeStruct((N, 128), _F32),
            jax.ShapeDtypeStruct((N, 128), _F32),
            jax.ShapeDtypeStruct((N, D), _F32),
        ],
    )(ni, na, wl1, w3f, whq8, wd816, wsc2)


def _pc1_body(SCAL, HID, es8_ref, w0_ref, w1_ref, w2_ref, h_ref):
    def layer(x, w_ref, fan_in):
        y = jnp.dot(x, w_ref[...], preferred_element_type=_F32) * _rs(fan_in)
        return y * jax.nn.sigmoid(y)
    h = layer(es8_ref[...], w0_ref, SCAL)
    h = layer(h, w1_ref, HID)
    h = layer(h, w2_ref, HID)
    h_ref[...] = h


def _pass_c1(es8, w0kv, w1kv, w2kv, SCAL, HID, BE8):
    E8 = es8.shape[0]
    grid = (E8 // BE8,)
    blk = pl.BlockSpec((BE8, 128), lambda i: (i, 0))
    full = pl.BlockSpec((128, 128), lambda i: (0, 0))
    return pl.pallas_call(
        functools.partial(_pc1_body, SCAL, HID),
        grid=grid,
        in_specs=[blk, full, full, full],
        out_specs=blk,
        out_shape=jax.ShapeDtypeStruct((E8, 128), _F32),
    )(es8, w0kv, w1kv, w2kv)


def _pc_body(D, HID, h_ref, usrc_ref, qdd_ref, ea_ref,
             v3_ref, r_ref, s_ref, m0_ref, x_ref, xmax_ref, macc_ref):
    h = h_ref[...]
    h3k = h[:, :HID]
    h3v = h[:, HID:2 * HID]
    wv = jnp.dot(h3v, v3_ref[...], preferred_element_type=_F32) * _rs(HID)
    ea = ea_ref[...]
    m0_ref[...] = wv * ea
    s = usrc_ref[...] * jnp.dot(h3k, r_ref[...], preferred_element_type=_F32)
    t16 = jnp.dot(s, s_ref[...], preferred_element_type=_F32)
    x = jnp.sum(t16 * qdd_ref[...], axis=1, keepdims=True)
    x = x * ea * (_rs(D) * _rs(HID))
    x_ref[...] = x

    @pl.when(pl.program_id(0) == 0)
    def _():
        macc_ref[0] = jnp.float32(-jnp.inf)

    m = jnp.maximum(macc_ref[0], jnp.max(x))
    macc_ref[0] = m
    xmax_ref[...] = jnp.full((1, 128), m, dtype=_F32)


def _pass_c(h3kv, usrc, qdd16, ea, v3, R, S16, D, HID, BE):
    E = h3kv.shape[0]
    grid = (E // BE,)
    full2 = lambda a, b: pl.BlockSpec((a, b), lambda i: (0, 0))
    return pl.pallas_call(
        functools.partial(_pc_body, D, HID),
        grid=grid,
        in_specs=[
            pl.BlockSpec((BE, 16), lambda i: (i, 0)),
            pl.BlockSpec((BE, 128), lambda i: (i, 0)),
            pl.BlockSpec((BE, 128), lambda i: (i, 0)),
            pl.BlockSpec((BE, 1), lambda i: (i, 0)),
            full2(HID, D),
            full2(HID, 128), full2(128, 128),
        ],
        out_specs=[
            pl.BlockSpec((BE, D), lambda i: (i, 0)),
            pl.BlockSpec((BE, 1), lambda i: (i, 0)),
            pl.BlockSpec((1, 128), lambda i: (0, 0)),
        ],
        out_shape=[
            jax.ShapeDtypeStruct((E, D), _F32),
            jax.ShapeDtypeStruct((E, 1), _F32),
            jax.ShapeDtypeStruct((1, 128), _F32),
        ],
        scratch_shapes=[pltpu.SMEM((1,), _F32)],
    )(h3kv, usrc, qdd16, ea, v3, R, S16)


def _pg_body(D, sc_ref, nf0_ref, nf1_ref, z0_ref, z1_ref, wl2_ref, out_ref):
    zs = z0_ref[...] + z1_ref[...]
    zs = jnp.where(zs == 0.0, jnp.ones_like(zs), zs)
    nf = (nf0_ref[...] + nf1_ref[...]) / zs
    out_ref[...] = sc_ref[...] + jnp.dot(
        nf, wl2_ref[...], preferred_element_type=_F32) * _rs(D)


def _pass_g(sc, nf0, nf1, z0, z1, wl2, BN):
    N, D = sc.shape
    grid = (N // BN,)
    blk = pl.BlockSpec((BN, D), lambda i: (i, 0))
    col = pl.BlockSpec((BN, 1), lambda i: (i, 0))
    return pl.pallas_call(
        functools.partial(_pg_body, D),
        grid=grid,
        in_specs=[blk, blk, blk, col, col,
                  pl.BlockSpec((D, D), lambda i: (0, 0))],
        out_specs=blk,
        out_shape=jax.ShapeDtypeStruct((N, D), _F32),
    )(sc, nf0, nf1, z0, z1, wl2)


_CH = 128
_NW = 32
_SC_PARAMS = pltpu.CompilerParams(needs_layout_passes=False)


def _worker_id():
    return lax.axis_index("s") * 2 + lax.axis_index("c")


def _n_chunks(total_chunks, wid):
    base = total_chunks // _NW
    rem = total_chunks % _NW
    return jnp.where(wid < rem, base + 1, base).astype(jnp.int32)


def _pass_b(u, qd, src, dst):
    N, HQ = u.shape
    E = src.shape[0]
    total_chunks = E // _CH
    mesh = plsc.VectorSubcoreMesh(core_axis_name="c", subcore_axis_name="s")

    @functools.partial(
        pl.kernel,
        out_type=(
            jax.ShapeDtypeStruct((E, HQ), _F32),
            jax.ShapeDtypeStruct((E, HQ), _F32),
        ),
        mesh=mesh,
        compiler_params=_SC_PARAMS,
        scratch_types=[
            pltpu.VMEM((2, _CH), jnp.int32),
            pltpu.VMEM((2, _CH), jnp.int32),
            pltpu.VMEM((2, _CH, HQ), _F32),
            pltpu.VMEM((2, _CH, HQ), _F32),
        ] + [pltpu.SemaphoreType.DMA] * 8,
    )
    def kfn(u_hbm, qd_hbm, src_hbm, dst_hbm, usrc_hbm, qdd_hbm,
            sidx, didx, urows, qrows,
            ss0, ss1, sd0, sd1, su0, su1, sq0, sq1):
        wid = _worker_id()
        nchunks = _n_chunks(total_chunks, wid)
        ssem = [ss0, ss1]
        dsem = [sd0, sd1]
        usem = [su0, su1]
        qsem = [sq0, sq1]

        def pair(jj, carry):
            idx_cp = [None, None]
            for b in range(2):
                ci = jj * 2 + b

                @pl.when(ci < nchunks)
                def _():
                    base = (wid + _NW * ci) * _CH
                    pltpu.async_copy(src_hbm.at[pl.ds(base, _CH)],
                                     sidx.at[b], ssem[b])
                    pltpu.async_copy(dst_hbm.at[pl.ds(base, _CH)],
                                     didx.at[b], dsem[b])
            gat_cp = [None, None]
            for b in range(2):
                ci = jj * 2 + b

                @pl.when(ci < nchunks)
                def _():
                    pltpu.make_async_copy(src_hbm.at[pl.ds(0, _CH)],
                                          sidx.at[b], ssem[b]).wait()
                    pltpu.make_async_copy(dst_hbm.at[pl.ds(0, _CH)],
                                          didx.at[b], dsem[b]).wait()
                    pltpu.async_copy(u_hbm.at[sidx.at[b]], urows.at[b],
                                     usem[b])
                    pltpu.async_copy(qd_hbm.at[didx.at[b]], qrows.at[b],
                                     qsem[b])
            for b in range(2):
                ci = jj * 2 + b

                @pl.when(ci < nchunks)
                def _():
                    base = (wid + _NW * ci) * _CH
                    pltpu.make_async_copy(u_hbm.at[pl.ds(0, _CH)],
                                          urows.at[b], usem[b]).wait()
                    pltpu.make_async_copy(qd_hbm.at[pl.ds(0, _CH)],
                                          qrows.at[b], qsem[b]).wait()
                    pltpu.async_copy(urows.at[b],
                                     usrc_hbm.at[pl.ds(base, _CH)], usem[b])
                    pltpu.async_copy(qrows.at[b],
                                     qdd_hbm.at[pl.ds(base, _CH)], qsem[b])
            for b in range(2):
                ci = jj * 2 + b

                @pl.when(ci < nchunks)
                def _():
                    base = (wid + _NW * ci) * _CH
                    pltpu.make_async_copy(urows.at[b],
                                          usrc_hbm.at[pl.ds(base, _CH)],
                                          usem[b]).wait()
                    pltpu.make_async_copy(qrows.at[b],
                                          qdd_hbm.at[pl.ds(base, _CH)],
                                          qsem[b]).wait()
            return carry

        lax.fori_loop(0, (total_chunks // _NW + 2) // 2, pair, 0)

    return kfn(u, qd, src, dst)


def _pass_f(feat, src, dst, x, xmax, m0, NP):
    N, D = feat.shape
    E = src.shape[0]
    total_chunks = E // _CH
    rows_per_tile = NP // 16
    nvec = D // 16
    mesh = plsc.VectorSubcoreMesh(core_axis_name="c", subcore_axis_name="s")

    @functools.partial(
        pl.kernel,
        out_type=(
            jax.ShapeDtypeStruct((2, NP, D), _F32),
            jax.ShapeDtypeStruct((2, NP), _F32),
        ),
        mesh=mesh,
        compiler_params=_SC_PARAMS,
        scratch_types=[
            pltpu.VMEM((_CH,), jnp.int32),
            pltpu.VMEM((_CH,), jnp.int32),
            pltpu.VMEM((_CH,), _F32),
            pltpu.VMEM((_CH,), _F32),
            pltpu.VMEM((128,), _F32),
            pltpu.VMEM((_CH, D), _F32),
            pltpu.VMEM((_CH, D), _F32),
            pltpu.VMEM((1024,), _F32),
            pltpu.VMEM_SHARED((NP, D), _F32),
            pltpu.VMEM_SHARED((NP,), _F32),
            pltpu.SemaphoreType.DMA,
            pltpu.SemaphoreType.DMA,
            pltpu.SemaphoreType.DMA,
            pltpu.SemaphoreType.DMA,
            pltpu.SemaphoreType.DMA,
        ],
    )
    def kfn(feat_hbm, src_hbm, dst_hbm, x_hbm, xmax_hbm, m0_hbm,
            nf_hbm, z_hbm,
            sidx, didx, xc, ec, xmv, featg, m0c, zb, nfsp, zsp,
            semA, semB, semC, semD, semE):
        cid = lax.axis_index("c")
        sid = lax.axis_index("s")
        wid = _worker_id()
        nchunks = _n_chunks(total_chunks, wid)

        pltpu.sync_copy(xmax_hbm, xmv)
        gmv = xmv[pl.ds(0, 16)]

        def zfill(i, carry):
            for v in range(nvec):
                featg[i, pl.ds(v * 16, 16)] = jnp.zeros((16,), _F32)
            return carry
        lax.fori_loop(0, _CH, zfill, 0)
        for v in range(1024 // 16):
            zb[pl.ds(v * 16, 16)] = jnp.zeros((16,), _F32)

        def zrows(i, carry):
            pltpu.sync_copy(
                featg, nfsp.at[pl.ds(sid * rows_per_tile + i * _CH, _CH)])
            return carry
        lax.fori_loop(0, rows_per_tile // _CH, zrows, 0)
        pltpu.sync_copy(zb.at[pl.ds(0, rows_per_tile)],
                        zsp.at[pl.ds(sid * rows_per_tile, rows_per_tile)])
        plsc.subcore_barrier()

        def body(i, carry):
            base = (wid + _NW * i) * _CH
            cp_s = pltpu.async_copy(src_hbm.at[pl.ds(base, _CH)], sidx, semA)
            cp_d = pltpu.async_copy(dst_hbm.at[pl.ds(base, _CH)], didx, semB)
            cp_x = pltpu.async_copy(x_hbm.at[pl.ds(base, _CH)], xc, semC)
            cp_m = pltpu.async_copy(m0_hbm.at[pl.ds(base, _CH)], m0c, semD)
            cp_s.wait()
            cp_g = pltpu.async_copy(feat_hbm.at[sidx], featg, semE)
            cp_x.wait()
            for v in range(_CH // 16):
                sl = pl.ds(v * 16, 16)
                ec[sl] = jnp.exp(xc[sl] - gmv)
            cp_d.wait()
            pltpu.sync_copy(ec, zsp.at[didx], add=True)
            cp_m.wait()
            cp_g.wait()

            def rows(jj, carry2):
                for uu in range(4):
                    j = jj * 4 + uu
                    av = plsc.load_gather(
                        ec, [jnp.full((16,), j, jnp.int32)])
                    for v in range(nvec):
                        sl = pl.ds(v * 16, 16)
                        m0c[j, sl] = m0c[j, sl] * featg[j, sl] * av
                return carry2
            lax.fori_loop(0, _CH // 4, rows, 0)

            pltpu.sync_copy(m0c, nfsp.at[didx], add=True)
            return carry

        lax.fori_loop(0, nchunks, body, 0)
        plsc.subcore_barrier()

        def dbody(i, carry):
            r0 = sid * rows_per_tile + i * _CH
            pltpu.sync_copy(nfsp.at[pl.ds(r0, _CH)], featg)
            pltpu.sync_copy(featg, nf_hbm.at[cid, pl.ds(r0, _CH)])
            return carry
        lax.fori_loop(0, rows_per_tile // _CH, dbody, 0)
        pltpu.sync_copy(zsp.at[pl.ds(sid * rows_per_tile, rows_per_tile)],
                        zb.at[pl.ds(0, rows_per_tile)])
        pltpu.sync_copy(zb.at[pl.ds(0, rows_per_tile)],
                        z_hbm.at[cid, pl.ds(sid * rows_per_tile,
                                            rows_per_tile)])

    return kfn(feat, src, dst, x, xmax, m0)


def kernel(node_input, node_attr, edge_src, edge_dst, edge_attr, edge_scalars,
           W_sc, W_lin1, W_hq, W_dot, W_lin2,
           fck_W0, fck_W1, fck_W2, fck_W3,
           fc_W0, fc_W1, fc_W2, fc_W3):
    N, D = node_input.shape
    E = edge_src.shape[0]
    QK = W_hq.shape[1]
    HID = fck_W0.shape[1]
    NP = ((N + 1023) // 1024) * 1024
    BN = 2000 if N % 2000 == 0 else 8
    BE = 4000 if E % 4000 == 0 else 128

    w3f = jnp.transpose(fck_W3.reshape(HID, D, QK), (1, 0, 2)).reshape(
        D, HID * QK)
    w3f = jnp.pad(w3f, ((0, 0), (0, 128 - HID * QK)))
    whq8 = jnp.pad(W_hq, ((0, 0), (0, 8 - QK)))
    wd816 = jnp.pad(W_dot[:, :, 0], ((0, 8 - QK), (0, 128 - QK)))
    wsc2 = W_sc[:, 0, :]
    R = jnp.pad(jnp.repeat(jnp.eye(HID, dtype=_F32), QK, axis=1),
                ((0, 0), (0, 128 - HID * QK)))
    S16 = jnp.pad(jnp.tile(jnp.eye(QK, dtype=_F32), (HID, 1)),
                  ((0, HID * QK - 32 + 96), (0, 128 - QK)))
    src = edge_src.astype(jnp.int32)
    dst = edge_dst.astype(jnp.int32)
    SCAL = edge_scalars.shape[1]
    b0 = jnp.concatenate([fck_W0, fc_W0], axis=1)
    b1 = jnp.concatenate([
        jnp.concatenate([fck_W1, jnp.zeros((HID, HID), _F32)], axis=1),
        jnp.concatenate([jnp.zeros((HID, HID), _F32), fc_W1], axis=1)],
        axis=0)
    b2 = jnp.concatenate([
        jnp.concatenate([fck_W2, jnp.zeros((HID, HID), _F32)], axis=1),
        jnp.concatenate([jnp.zeros((HID, HID), _F32), fc_W2], axis=1)],
        axis=0)
    eye8 = jnp.eye(128 // SCAL, dtype=_F32)
    w0kv = jnp.kron(eye8, b0)
    w1kv = jnp.kron(eye8, b1)
    w2kv = jnp.kron(eye8, b2)
    es8 = edge_scalars.reshape(E * SCAL // 128, 128)

    feat, u, qd16, sc = _pass_a(node_input, node_attr, W_lin1, w3f, whq8,
                                wd816, wsc2, BN)
    usrc, qdd16 = _pass_b(u, qd16, src, dst)
    h3kv8 = _pass_c1(es8, w0kv, w1kv, w2kv, SCAL, HID, 2000)
    h3kv = h3kv8.reshape(E, 2 * HID)
    m0, x, xmax = _pass_c(h3kv, usrc, qdd16, edge_attr, fc_W3, R, S16,
                          D, HID, BE)
    nf2, z2 = _pass_f(feat, src, dst, x.reshape(E), xmax.reshape(128),
                      m0, NP)
    z0 = z2[0, :N].reshape(N, 1)
    z1 = z2[1, :N].reshape(N, 1)
    out = _pass_g(sc, nf2[0, :N, :], nf2[1, :N, :], z0, z1, W_lin2, BN)
    return out
